# Initial kernel scaffold; baseline (speedup 1.0000x reference)
import jax
import jax.numpy as jnp
from jax.experimental import pallas as pl


def kernel(molecule_atoms, W, N):
    raise NotImplementedError("write your pallas kernel here")


# speedup vs baseline: 2.6011x; 2.6011x over previous
import jax
import jax.numpy as jnp
from jax.experimental import pallas as pl
from jax.experimental.pallas import tpu as pltpu

_B, _A, _F = 1024, 128, 75
_C = 16
_H = 33
_OUT = 64
_ROWS = _B * _A


def _body(n_ref, x_ref, w_ref, o_ref):
    rblk = x_ref.shape[0]
    x = x_ref[...]
    cls = x[:, 0:_C]
    ids = jax.lax.broadcasted_iota(jnp.int32, (rblk, _C), 1)
    m = jnp.max(cls, axis=1, keepdims=True)
    cand = jnp.where(cls == m, ids, _C)
    p = jnp.min(cand, axis=1, keepdims=True)
    onehot = (ids == p).astype(jnp.float32)
    com = jnp.dot(onehot, w_ref[...],
                  preferred_element_type=jnp.float32,
                  precision=jax.lax.Precision.HIGHEST)
    oth = x[:, 44:_F]
    norm = jnp.sqrt(jnp.sum(oth * oth, axis=1, keepdims=True))
    tf = oth / jnp.maximum(norm, 1e-12)
    out = jnp.concatenate([com, tf], axis=1)
    row0 = pl.program_id(0) * rblk
    rows = row0 + jax.lax.broadcasted_iota(jnp.int32, (rblk, 1), 0)
    valid = rows < n_ref[0] * _A
    o_ref[...] = jnp.where(valid, out, 0.0)


def kernel(molecule_atoms, W, N):
    x2 = molecule_atoms.reshape(_ROWS, _F)
    n_arr = jnp.asarray(N, jnp.int32).reshape(1)
    rblk = 8192
    grid = _ROWS // rblk
    out = pl.pallas_call(
        _body,
        grid=(grid,),
        in_specs=[
            pl.BlockSpec(memory_space=pltpu.SMEM),
            pl.BlockSpec((rblk, _F), lambda i: (i, 0)),
            pl.BlockSpec((_C, _H), lambda i: (0, 0)),
        ],
        out_specs=pl.BlockSpec((rblk, _OUT), lambda i: (i, 0)),
        out_shape=jax.ShapeDtypeStruct((_ROWS, _OUT), jnp.float32),
    )(n_arr, x2, W)
    return out.reshape(_B, _A, _OUT)
